# Initial kernel scaffold; baseline (speedup 1.0000x reference)
import functools

import jax
import jax.numpy as jnp
from jax import lax
from jax.experimental import pallas as pl
from jax.experimental.pallas import tpu as pltpu
from jax.experimental.pallas import tpu_sc as plsc

_NC = 2
_NS = 16
_NW = _NC * _NS
_CH = 128


def _sc_gather(user_table, item_table, uidx, iidx):
  n_ch = uidx.shape[1]
  b_per_w = n_ch * _CH
  B = _NW * b_per_w
  du = user_table.shape[1]
  di = item_table.shape[1]

  mesh = plsc.VectorSubcoreMesh(core_axis_name="c", subcore_axis_name="s")

  @functools.partial(
      pl.kernel,
      out_type=[
          jax.ShapeDtypeStruct((B, du), jnp.float32),
          jax.ShapeDtypeStruct((B, di), jnp.float32),
      ],
      mesh=mesh,
      scratch_types=[
          pltpu.VMEM((n_ch, _CH), jnp.int32),
          pltpu.VMEM((n_ch, _CH), jnp.int32),
          pltpu.VMEM((b_per_w, du), jnp.float32),
          pltpu.VMEM((b_per_w, di), jnp.float32),
          pltpu.SemaphoreType.DMA,
          pltpu.SemaphoreType.DMA,
      ],
  )
  def k(ut, it, ui, ii, uo, io, ui_v, ii_v, ur_v, ir_v, su, si):
    c = lax.axis_index("c")
    s = lax.axis_index("s")
    wid = s * _NC + c
    pltpu.sync_copy(ui.at[wid], ui_v)
    pltpu.sync_copy(ii.at[wid], ii_v)
    cps = []
    for j in range(n_ch):
      cps.append(pltpu.async_copy(ut.at[ui_v.at[j]], ur_v.at[pl.ds(j * _CH, _CH)], su))
      cps.append(pltpu.async_copy(it.at[ii_v.at[j]], ir_v.at[pl.ds(j * _CH, _CH)], si))
    for cp in cps:
      cp.wait()
    base = wid * b_per_w
    pltpu.sync_copy(ur_v, uo.at[pl.ds(base, b_per_w)])
    pltpu.sync_copy(ir_v, io.at[pl.ds(base, b_per_w)])

  return k(user_table, item_table, uidx, iidx)


def _leaky(x):
  return jnp.where(x >= 0, x, 0.01 * x)


def _tc_mlp(user_emb, item_emb, feats, pg_idx, cg_idx, in_idx,
            pg_table, cg_table, in_table, Ws, bs, *, interpret=False):
  B = user_emb.shape[0]
  BM = 1024
  grid = (B // BM,)
  n_pg = pg_table.shape[0]
  n_cg = cg_table.shape[0]
  n_in = in_table.shape[0]

  def body(u_ref, i_ref, f_ref, pg_ref, cg_ref, in_ref,
           pgt_ref, cgt_ref, int_ref, *wb_refs):
    o_ref = wb_refs[-1]
    w_refs = wb_refs[0:6]
    b_refs = wb_refs[6:12]
    u = u_ref[...]
    it = i_ref[...]
    f = f_ref[...]
    oh_pg = (pg_ref[...] == lax.broadcasted_iota(jnp.int32, (1, n_pg), 1)
             ).astype(jnp.float32)
    oh_cg = (cg_ref[...] == lax.broadcasted_iota(jnp.int32, (1, n_cg), 1)
             ).astype(jnp.float32)
    oh_in = (in_ref[...] == lax.broadcasted_iota(jnp.int32, (1, n_in), 1)
             ).astype(jnp.float32)
    pgE = jnp.dot(oh_pg, pgt_ref[...], preferred_element_type=jnp.float32)
    cgE = jnp.dot(oh_cg, cgt_ref[...], preferred_element_type=jnp.float32)
    inE = jnp.dot(oh_in, int_ref[...], preferred_element_type=jnp.float32)
    x = jnp.concatenate([u, it, pgE, cgE, inE, f], axis=1)
    for wr, br in zip(w_refs, b_refs):
      x = jnp.dot(x, wr[...], preferred_element_type=jnp.float32) + br[...]
      x = _leaky(x)
    o_ref[...] = x

  def row_spec(d):
    return pl.BlockSpec((BM, d), lambda i: (i, 0))

  def full_spec(shape):
    nd = len(shape)
    if nd == 1:
      return pl.BlockSpec(shape, lambda i: (0,))
    return pl.BlockSpec(shape, lambda i: (0, 0))

  in_specs = [
      row_spec(user_emb.shape[1]),
      row_spec(item_emb.shape[1]),
      row_spec(feats.shape[1]),
      row_spec(1), row_spec(1), row_spec(1),
      full_spec(pg_table.shape), full_spec(cg_table.shape),
      full_spec(in_table.shape),
  ]
  for W in Ws:
    in_specs.append(full_spec(W.shape))
  for b in bs:
    in_specs.append(full_spec(b.shape))

  out_dim = Ws[-1].shape[1]
  return pl.pallas_call(
      body,
      grid=grid,
      in_specs=in_specs,
      out_specs=pl.BlockSpec((BM, out_dim), lambda i: (i, 0)),
      out_shape=jax.ShapeDtypeStruct((B, out_dim), jnp.float32),
      interpret=interpret,
  )(user_emb, item_emb, feats, pg_idx, cg_idx, in_idx,
    pg_table, cg_table, in_table, *Ws, *bs)


def kernel(user_input, item_input, prices, sales_channels, club_status,
           age_groups, product_groups, color_groups, index_name,
           user_table, item_table, pg_table, cg_table, in_table, Ws, bs):
  B = user_input.shape[0]
  n_ch = B // (_NW * _CH)
  uidx = user_input.astype(jnp.int32).reshape(_NW, n_ch, _CH)
  iidx = item_input.astype(jnp.int32).reshape(_NW, n_ch, _CH)
  user_emb, item_emb = _sc_gather(user_table, item_table, uidx, iidx)

  feats = jnp.stack([prices, sales_channels, club_status, age_groups], axis=1)
  pg = product_groups.astype(jnp.int32).reshape(B, 1)
  cg = color_groups.astype(jnp.int32).reshape(B, 1)
  inm = index_name.astype(jnp.int32).reshape(B, 1)
  return _tc_mlp(user_emb, item_emb, feats, pg, cg, inm,
                 pg_table, cg_table, in_table, Ws, bs)


# speedup vs baseline: 1.0352x; 1.0352x over previous
import functools

import jax
import jax.numpy as jnp
from jax import lax
from jax.experimental import pallas as pl
from jax.experimental.pallas import tpu as pltpu
from jax.experimental.pallas import tpu_sc as plsc

_NC = 2
_NS = 16
_NW = _NC * _NS
_CH = 128


def _sc_gather(user_table, item_table, uidx, iidx):
  n_ch = uidx.shape[1]
  b_per_w = n_ch * _CH
  B = _NW * b_per_w
  du = user_table.shape[1]
  di = item_table.shape[1]

  mesh = plsc.VectorSubcoreMesh(core_axis_name="c", subcore_axis_name="s")

  @functools.partial(
      pl.kernel,
      out_type=[
          jax.ShapeDtypeStruct((B, du), jnp.float32),
          jax.ShapeDtypeStruct((B, di), jnp.float32),
      ],
      mesh=mesh,
      scratch_types=[
          pltpu.VMEM((n_ch, _CH), jnp.int32),
          pltpu.VMEM((n_ch, _CH), jnp.int32),
          pltpu.VMEM((b_per_w, du), jnp.float32),
          pltpu.VMEM((b_per_w, di), jnp.float32),
          pltpu.SemaphoreType.DMA,
          pltpu.SemaphoreType.DMA,
      ],
      compiler_params=pltpu.CompilerParams(use_tc_tiling_on_sc=False),
  )
  def k(ut, it, ui, ii, uo, io, ui_v, ii_v, ur_v, ir_v, su, si):
    c = lax.axis_index("c")
    s = lax.axis_index("s")
    wid = s * _NC + c
    pltpu.sync_copy(ui.at[wid], ui_v)
    pltpu.sync_copy(ii.at[wid], ii_v)
    cps = []
    for j in range(n_ch):
      cps.append(pltpu.async_copy(ut.at[ui_v.at[j]], ur_v.at[pl.ds(j * _CH, _CH)], su))
      cps.append(pltpu.async_copy(it.at[ii_v.at[j]], ir_v.at[pl.ds(j * _CH, _CH)], si))
    for cp in cps:
      cp.wait()
    base = wid * b_per_w
    pltpu.sync_copy(ur_v, uo.at[pl.ds(base, b_per_w)])
    pltpu.sync_copy(ir_v, io.at[pl.ds(base, b_per_w)])

  return k(user_table, item_table, uidx, iidx)


def _leaky(x):
  return jnp.where(x >= 0, x, 0.01 * x)


def _tc_mlp(user_emb, item_emb, feats, pg_idx, cg_idx, in_idx,
            pg_table, cg_table, in_table, Ws, bs, *, interpret=False):
  B = user_emb.shape[0]
  BM = 1024
  grid = (B // BM,)
  n_pg = pg_table.shape[0]
  n_cg = cg_table.shape[0]
  n_in = in_table.shape[0]

  def body(u_ref, i_ref, f_ref, pg_ref, cg_ref, in_ref,
           pgt_ref, cgt_ref, int_ref, *wb_refs):
    o_ref = wb_refs[-1]
    w_refs = wb_refs[0:6]
    b_refs = wb_refs[6:12]
    u = u_ref[...]
    it = i_ref[...]
    f = f_ref[...]
    oh_pg = (pg_ref[...] == lax.broadcasted_iota(jnp.int32, (1, n_pg), 1)
             ).astype(jnp.float32)
    oh_cg = (cg_ref[...] == lax.broadcasted_iota(jnp.int32, (1, n_cg), 1)
             ).astype(jnp.float32)
    oh_in = (in_ref[...] == lax.broadcasted_iota(jnp.int32, (1, n_in), 1)
             ).astype(jnp.float32)
    pgE = jnp.dot(oh_pg, pgt_ref[...], preferred_element_type=jnp.float32)
    cgE = jnp.dot(oh_cg, cgt_ref[...], preferred_element_type=jnp.float32)
    inE = jnp.dot(oh_in, int_ref[...], preferred_element_type=jnp.float32)
    x = jnp.concatenate([u, it, pgE, cgE, inE, f], axis=1)
    for wr, br in zip(w_refs, b_refs):
      x = jnp.dot(x, wr[...], preferred_element_type=jnp.float32) + br[...]
      x = _leaky(x)
    o_ref[...] = x

  def row_spec(d):
    return pl.BlockSpec((BM, d), lambda i: (i, 0))

  def full_spec(shape):
    nd = len(shape)
    if nd == 1:
      return pl.BlockSpec(shape, lambda i: (0,))
    return pl.BlockSpec(shape, lambda i: (0, 0))

  in_specs = [
      row_spec(user_emb.shape[1]),
      row_spec(item_emb.shape[1]),
      row_spec(feats.shape[1]),
      row_spec(1), row_spec(1), row_spec(1),
      full_spec(pg_table.shape), full_spec(cg_table.shape),
      full_spec(in_table.shape),
  ]
  for W in Ws:
    in_specs.append(full_spec(W.shape))
  for b in bs:
    in_specs.append(full_spec(b.shape))

  out_dim = Ws[-1].shape[1]
  return pl.pallas_call(
      body,
      grid=grid,
      in_specs=in_specs,
      out_specs=pl.BlockSpec((BM, out_dim), lambda i: (i, 0)),
      out_shape=jax.ShapeDtypeStruct((B, out_dim), jnp.float32),
      interpret=interpret,
  )(user_emb, item_emb, feats, pg_idx, cg_idx, in_idx,
    pg_table, cg_table, in_table, *Ws, *bs)


def kernel(user_input, item_input, prices, sales_channels, club_status,
           age_groups, product_groups, color_groups, index_name,
           user_table, item_table, pg_table, cg_table, in_table, Ws, bs):
  B = user_input.shape[0]
  n_ch = B // (_NW * _CH)
  uidx = user_input.astype(jnp.int32).reshape(_NW, n_ch, _CH)
  iidx = item_input.astype(jnp.int32).reshape(_NW, n_ch, _CH)
  user_emb, item_emb = _sc_gather(user_table, item_table, uidx, iidx)

  feats = jnp.stack([prices, sales_channels, club_status, age_groups], axis=1)
  pg = product_groups.astype(jnp.int32).reshape(B, 1)
  cg = color_groups.astype(jnp.int32).reshape(B, 1)
  inm = index_name.astype(jnp.int32).reshape(B, 1)
  return _tc_mlp(user_emb, item_emb, feats, pg, cg, inm,
                 pg_table, cg_table, in_table, Ws, bs)
